# baseline (speedup 1.0000x reference)
import functools

import jax
import jax.numpy as jnp
from jax import lax
from jax.experimental import pallas as pl
from jax.experimental.pallas import tpu as pltpu
from jax.experimental.pallas import tpu_sc as plsc

_BATCH = 16384
_EMB = 64
_L = 16
_NC = 2
_NS = 16
_NW = _NC * _NS
_N_PER = _BATCH // _NW
_CHUNK = 128
_NCHUNK = _N_PER // _CHUNK


def _sc_body(u_hbm, v_hbm, table_hbm, out_hbm,
             uidx_v, vidx_v, umaj_v, vmaj_v, urows_v, vrows_v, out_v,
             sem_u, sem_v):
    wid = lax.axis_index("s") * _NC + lax.axis_index("c")
    base = wid * _N_PER

    pltpu.sync_copy(u_hbm.at[pl.ds(base, _N_PER)], uidx_v)
    pltpu.sync_copy(v_hbm.at[pl.ds(base, _N_PER)], vidx_v)

    for t in range(_N_PER // _L):
        sl = pl.ds(t * _L, _L)
        umaj_v[sl] = jnp.right_shift(uidx_v[sl], 1)
        vmaj_v[sl] = jnp.right_shift(vidx_v[sl], 1)

    lane = lax.iota(jnp.int32, _L)

    for c in range(_NCHUNK):
        csl = pl.ds(c * _CHUNK, _CHUNK)
        cu = pltpu.async_copy(table_hbm.at[umaj_v.at[csl]], urows_v, sem_u)
        cv = pltpu.async_copy(table_hbm.at[vmaj_v.at[csl]], vrows_v, sem_v)
        cu.wait()
        cv.wait()

        def group(g, carry):
            rsl = pl.ds(c * _CHUNK + g * _L, _L)
            lrows = g * _L + lane
            ucol0 = jnp.left_shift(jnp.bitwise_and(uidx_v[rsl], 1), 6)
            vcol0 = jnp.left_shift(jnp.bitwise_and(vidx_v[rsl], 1), 6)
            acc = jnp.zeros((_L,), jnp.float32)
            for j in range(_EMB):
                uu = plsc.load_gather(urows_v, [lrows, ucol0 + j])
                vv = plsc.load_gather(vrows_v, [lrows, vcol0 + j])
                acc = acc + uu * vv
            out_v[pl.ds(c * _CHUNK + g * _L, _L)] = 1.0 / (1.0 + jnp.exp(-acc))
            return carry

        lax.fori_loop(0, _CHUNK // _L, group, 0)

    pltpu.sync_copy(out_v, out_hbm.at[pl.ds(base, _N_PER)])


@jax.jit
def _dist_mult(u, v, table2):
    mesh = plsc.VectorSubcoreMesh(
        core_axis_name="c", subcore_axis_name="s",
        num_cores=_NC, num_subcores=_NS)
    run = pl.kernel(
        _sc_body,
        out_type=jax.ShapeDtypeStruct((_BATCH,), jnp.float32),
        mesh=mesh,
        scratch_types=[
            pltpu.VMEM((_N_PER,), jnp.int32),
            pltpu.VMEM((_N_PER,), jnp.int32),
            pltpu.VMEM((_N_PER,), jnp.int32),
            pltpu.VMEM((_N_PER,), jnp.int32),
            pltpu.VMEM((_CHUNK, 2 * _EMB), jnp.float32),
            pltpu.VMEM((_CHUNK, 2 * _EMB), jnp.float32),
            pltpu.VMEM((_N_PER,), jnp.float32),
            pltpu.SemaphoreType.DMA,
            pltpu.SemaphoreType.DMA,
        ],
        compiler_params=pltpu.CompilerParams(
            needs_layout_passes=False, use_tc_tiling_on_sc=True),
    )
    return run(u, v, table2)


def kernel(u, v, emb_weight):
    n = emb_weight.shape[0]
    table2 = emb_weight.reshape(n // 2, 2 * _EMB)
    return _dist_mult(u.astype(jnp.int32), v.astype(jnp.int32), table2)


# speedup vs baseline: 2.3423x; 2.3423x over previous
import functools

import jax
import jax.numpy as jnp
from jax import lax
from jax.experimental import pallas as pl
from jax.experimental.pallas import tpu as pltpu
from jax.experimental.pallas import tpu_sc as plsc

_BATCH = 16384
_EMB = 64
_L = 16
_NC = 2
_NS = 16
_NW = _NC * _NS
_N_PER = _BATCH // _NW


def _sc_body(u_hbm, v_hbm, table_hbm, out_hbm,
             uidx_v, vidx_v, ublk, vblk, out_v, sem_u, sem_v):
    wid = lax.axis_index("s") * _NC + lax.axis_index("c")
    base = wid * _N_PER

    pltpu.sync_copy(u_hbm.at[pl.ds(base, _N_PER)], uidx_v)
    pltpu.sync_copy(v_hbm.at[pl.ds(base, _N_PER)], vidx_v)

    lane = lax.iota(jnp.int32, _L)

    def group(g, carry):
        sl = pl.ds(g * _L, _L)
        uvec = uidx_v[sl]
        vvec = vidx_v[sl]
        ubase = jnp.bitwise_and(uvec, ~7)
        vbase = jnp.bitwise_and(vvec, ~7)
        for k in range(_L):
            ub = pl.multiple_of(ubase[k], 8)
            vb = pl.multiple_of(vbase[k], 8)
            pltpu.async_copy(table_hbm.at[pl.ds(ub, 8), :],
                             ublk.at[k], sem_u)
            pltpu.async_copy(table_hbm.at[pl.ds(vb, 8), :],
                             vblk.at[k], sem_v)
        for k in range(_L):
            pltpu.make_async_copy(table_hbm.at[pl.ds(0, 8), :],
                                  ublk.at[k], sem_u).wait()
            pltpu.make_async_copy(table_hbm.at[pl.ds(0, 8), :],
                                  vblk.at[k], sem_v).wait()
        uoff = jnp.bitwise_and(uvec, 7)
        voff = jnp.bitwise_and(vvec, 7)
        acc = jnp.zeros((_L,), jnp.float32)
        for j in range(_EMB):
            jc = jnp.full((_L,), j, jnp.int32)
            uu = plsc.load_gather(ublk, [lane, uoff, jc])
            vv = plsc.load_gather(vblk, [lane, voff, jc])
            acc = acc + uu * vv
        out_v[sl] = 1.0 / (1.0 + jnp.exp(-acc))
        return carry

    lax.fori_loop(0, _N_PER // _L, group, 0)

    pltpu.sync_copy(out_v, out_hbm.at[pl.ds(base, _N_PER)])


@jax.jit
def _dist_mult(u, v, emb_weight):
    mesh = plsc.VectorSubcoreMesh(
        core_axis_name="c", subcore_axis_name="s",
        num_cores=_NC, num_subcores=_NS)
    run = pl.kernel(
        _sc_body,
        out_type=jax.ShapeDtypeStruct((_BATCH,), jnp.float32),
        mesh=mesh,
        scratch_types=[
            pltpu.VMEM((_N_PER,), jnp.int32),
            pltpu.VMEM((_N_PER,), jnp.int32),
            pltpu.VMEM((_L, 8, _EMB), jnp.float32),
            pltpu.VMEM((_L, 8, _EMB), jnp.float32),
            pltpu.VMEM((_N_PER,), jnp.float32),
            pltpu.SemaphoreType.DMA,
            pltpu.SemaphoreType.DMA,
        ],
        compiler_params=pltpu.CompilerParams(
            needs_layout_passes=False, use_tc_tiling_on_sc=True),
    )
    return run(u, v, emb_weight)


def kernel(u, v, emb_weight):
    return _dist_mult(u.astype(jnp.int32), v.astype(jnp.int32), emb_weight)
